# baseline (device time: 96589 ns/iter reference)
import jax
import jax.numpy as jnp
from jax import lax
from jax.experimental import pallas as pl
from jax.experimental.pallas import tpu as pltpu

N_DEV = 16
N_HOP = N_DEV - 1
M_BLK = 128
N_COL = 2048
N_HALF = N_COL // 2
SUB = 2
SLOTS = 3
N_SUB = N_HALF // SUB


def kernel(x, w_mat):
    def body(x_ref, w_ref, out_ref,
             send_a, recv_a, send_b, recv_b,
             send_sems_a, recv_sems_a, send_sems_b, recv_sems_b,
             credit_a, credit_b):
        me = lax.axis_index("i")
        left = (me - 1) % N_DEV
        right = (me + 1) % N_DEV

        barrier_sem = pltpu.get_barrier_semaphore()
        for nbr in [left, right]:
            pl.semaphore_signal(
                barrier_sem, inc=1,
                device_id=(nbr,), device_id_type=pl.DeviceIdType.MESH,
            )
        pl.semaphore_wait(barrier_sem, 2)

        rings = {
            "a": dict(send=send_a, recv=recv_a, ssem=send_sems_a,
                      rsem=recv_sems_a, credit=credit_a, to=right,
                      frm=left, col0=0),
            "b": dict(send=send_b, recv=recv_b, ssem=send_sems_b,
                      rsem=recv_sems_b, credit=credit_b, to=left,
                      frm=right, col0=N_HALF),
        }

        def partial(c, ring, s):
            xc = x_ref[pl.ds(c * M_BLK, M_BLK), :]
            lo = rings[ring]["col0"] + s * N_SUB
            return jnp.dot(xc, w_ref[:, lo:lo + N_SUB],
                           preferred_element_type=jnp.float32)

        def c_send(t, ring):
            return (me - 1 - t) % N_DEV if ring == "a" else (me + 1 + t) % N_DEV

        def silu(y):
            return y * jax.nn.sigmoid(y)

        rdmas = {}

        def start_send(t, ring, s):
            r = rings[ring]
            if t >= SLOTS:
                pl.semaphore_wait(r["credit"].at[s], 1)
            d = pltpu.make_async_remote_copy(
                src_ref=r["send"].at[t % SLOTS, s],
                dst_ref=r["recv"].at[t % SLOTS, s],
                send_sem=r["ssem"].at[t % SLOTS, s],
                recv_sem=r["rsem"].at[t % SLOTS, s],
                device_id=(r["to"],),
                device_id_type=pl.DeviceIdType.MESH,
            )
            rdmas[(t, ring, s)] = d
            d.start()

        for ring in ("a", "b"):
            for s in range(SUB):
                rings[ring]["send"][0, s, :, :] = partial(c_send(0, ring), ring, s)
        for s in range(SUB):
            for ring in ("a", "b"):
                start_send(0, ring, s)

        unwaited_sends = set(rdmas.keys())

        for t in range(N_HOP):
            for s in range(SUB):
                for ring in ("a", "b"):
                    r = rings[ring]
                    c_in = c_send(t + 1, ring)
                    p = partial(c_in, ring, s)
                    rdmas[(t, ring, s)].wait_recv()
                    acc = r["recv"][t % SLOTS, s, :, :] + p
                    if t < N_HOP - 1:
                        if t + 1 - SLOTS >= 0:
                            key = (t + 1 - SLOTS, ring, s)
                            rdmas[key].wait_send()
                            unwaited_sends.discard(key)
                        r["send"][(t + 1) % SLOTS, s, :, :] = acc
                        if t <= N_HOP - 1 - SLOTS:
                            pl.semaphore_signal(
                                r["credit"].at[s], inc=1,
                                device_id=(r["frm"],),
                                device_id_type=pl.DeviceIdType.MESH,
                            )
                        start_send(t + 1, ring, s)
                        unwaited_sends.add((t + 1, ring, s))
                    else:
                        lo = r["col0"] + s * N_SUB
                        out_ref[:, lo:lo + N_SUB] = silu(acc)

        for key in sorted(unwaited_sends):
            rdmas[key].wait_send()

    return pl.pallas_call(
        body,
        out_shape=jax.ShapeDtypeStruct((M_BLK, N_COL), jnp.float32),
        in_specs=[
            pl.BlockSpec(memory_space=pltpu.VMEM),
            pl.BlockSpec(memory_space=pltpu.VMEM),
        ],
        out_specs=pl.BlockSpec(memory_space=pltpu.VMEM),
        scratch_shapes=[
            pltpu.VMEM((SLOTS, SUB, M_BLK, N_SUB), jnp.float32),
            pltpu.VMEM((SLOTS, SUB, M_BLK, N_SUB), jnp.float32),
            pltpu.VMEM((SLOTS, SUB, M_BLK, N_SUB), jnp.float32),
            pltpu.VMEM((SLOTS, SUB, M_BLK, N_SUB), jnp.float32),
            pltpu.SemaphoreType.DMA((SLOTS, SUB)),
            pltpu.SemaphoreType.DMA((SLOTS, SUB)),
            pltpu.SemaphoreType.DMA((SLOTS, SUB)),
            pltpu.SemaphoreType.DMA((SLOTS, SUB)),
            pltpu.SemaphoreType.REGULAR((SUB,)),
            pltpu.SemaphoreType.REGULAR((SUB,)),
        ],
        compiler_params=pltpu.CompilerParams(collective_id=0),
    )(x, w_mat)


# device time: 95727 ns/iter; 1.0090x vs baseline; 1.0090x over previous
import jax
import jax.numpy as jnp
from jax import lax
from jax.experimental import pallas as pl
from jax.experimental.pallas import tpu as pltpu

N_DEV = 16
N_HOP = N_DEV - 1
M_BLK = 128
N_COL = 2048
N_HALF = N_COL // 2
SUB = 4
SLOTS = 2
N_SUB = N_HALF // SUB


def kernel(x, w_mat):
    def body(x_ref, w_ref, out_ref,
             send_a, recv_a, send_b, recv_b,
             send_sems_a, recv_sems_a, send_sems_b, recv_sems_b,
             credit_a, credit_b):
        me = lax.axis_index("i")
        left = (me - 1) % N_DEV
        right = (me + 1) % N_DEV

        barrier_sem = pltpu.get_barrier_semaphore()
        for nbr in [left, right]:
            pl.semaphore_signal(
                barrier_sem, inc=1,
                device_id=(nbr,), device_id_type=pl.DeviceIdType.MESH,
            )
        pl.semaphore_wait(barrier_sem, 2)

        rings = {
            "a": dict(send=send_a, recv=recv_a, ssem=send_sems_a,
                      rsem=recv_sems_a, credit=credit_a, to=right,
                      frm=left, col0=0),
            "b": dict(send=send_b, recv=recv_b, ssem=send_sems_b,
                      rsem=recv_sems_b, credit=credit_b, to=left,
                      frm=right, col0=N_HALF),
        }

        def partial(c, ring, s):
            xc = x_ref[pl.ds(c * M_BLK, M_BLK), :]
            lo = rings[ring]["col0"] + s * N_SUB
            return jnp.dot(xc, w_ref[:, lo:lo + N_SUB],
                           preferred_element_type=jnp.float32)

        def c_send(t, ring):
            return (me - 1 - t) % N_DEV if ring == "a" else (me + 1 + t) % N_DEV

        def silu(y):
            return y * jax.nn.sigmoid(y)

        rdmas = {}

        def start_send(t, ring, s):
            r = rings[ring]
            if t >= SLOTS:
                pl.semaphore_wait(r["credit"].at[s], 1)
            d = pltpu.make_async_remote_copy(
                src_ref=r["send"].at[t % SLOTS, s],
                dst_ref=r["recv"].at[t % SLOTS, s],
                send_sem=r["ssem"].at[t % SLOTS, s],
                recv_sem=r["rsem"].at[t % SLOTS, s],
                device_id=(r["to"],),
                device_id_type=pl.DeviceIdType.MESH,
            )
            rdmas[(t, ring, s)] = d
            d.start()

        for ring in ("a", "b"):
            for s in range(SUB):
                rings[ring]["send"][0, s, :, :] = partial(c_send(0, ring), ring, s)
        for s in range(SUB):
            for ring in ("a", "b"):
                start_send(0, ring, s)

        unwaited_sends = set(rdmas.keys())

        for t in range(N_HOP):
            for s in range(SUB):
                for ring in ("a", "b"):
                    r = rings[ring]
                    c_in = c_send(t + 1, ring)
                    p = partial(c_in, ring, s)
                    rdmas[(t, ring, s)].wait_recv()
                    acc = r["recv"][t % SLOTS, s, :, :] + p
                    if t < N_HOP - 1:
                        if t + 1 - SLOTS >= 0:
                            key = (t + 1 - SLOTS, ring, s)
                            rdmas[key].wait_send()
                            unwaited_sends.discard(key)
                        r["send"][(t + 1) % SLOTS, s, :, :] = acc
                        if t <= N_HOP - 1 - SLOTS:
                            pl.semaphore_signal(
                                r["credit"].at[s], inc=1,
                                device_id=(r["frm"],),
                                device_id_type=pl.DeviceIdType.MESH,
                            )
                        start_send(t + 1, ring, s)
                        unwaited_sends.add((t + 1, ring, s))
                    else:
                        lo = r["col0"] + s * N_SUB
                        out_ref[:, lo:lo + N_SUB] = silu(acc)

        for key in sorted(unwaited_sends):
            rdmas[key].wait_send()

    return pl.pallas_call(
        body,
        out_shape=jax.ShapeDtypeStruct((M_BLK, N_COL), jnp.float32),
        in_specs=[
            pl.BlockSpec(memory_space=pltpu.VMEM),
            pl.BlockSpec(memory_space=pltpu.VMEM),
        ],
        out_specs=pl.BlockSpec(memory_space=pltpu.VMEM),
        scratch_shapes=[
            pltpu.VMEM((SLOTS, SUB, M_BLK, N_SUB), jnp.float32),
            pltpu.VMEM((SLOTS, SUB, M_BLK, N_SUB), jnp.float32),
            pltpu.VMEM((SLOTS, SUB, M_BLK, N_SUB), jnp.float32),
            pltpu.VMEM((SLOTS, SUB, M_BLK, N_SUB), jnp.float32),
            pltpu.SemaphoreType.DMA((SLOTS, SUB)),
            pltpu.SemaphoreType.DMA((SLOTS, SUB)),
            pltpu.SemaphoreType.DMA((SLOTS, SUB)),
            pltpu.SemaphoreType.DMA((SLOTS, SUB)),
            pltpu.SemaphoreType.REGULAR((SUB,)),
            pltpu.SemaphoreType.REGULAR((SUB,)),
        ],
        compiler_params=pltpu.CompilerParams(collective_id=0),
    )(x, w_mat)
